# baseline (speedup 1.0000x reference)
import functools

import jax
import jax.numpy as jnp
from jax import lax
from jax.experimental import pallas as pl
from jax.experimental.pallas import tpu as pltpu
from jax.experimental.pallas import tpu_sc as plsc

_Q = 0.15
_INF_KEY = 2139095041
_NC = 2
_NS = 16
_CPB = 8
_G = 16


def _score_select_kernel(h_ref, m_ref, sel_ref, kf_ref, keys_s, *, nb, nt, tb, t):
    b = pl.program_id(0)
    ti = pl.program_id(1)
    x = h_ref[0]
    d = x.shape[1]
    y = x * x
    acc = y[:, 0:128]
    for i in range(1, d // 128):
        acc = acc + y[:, i * 128:(i + 1) * 128]
    v = acc.T
    s = v[0:8]
    for k in range(1, 16):
        s = s + v[8 * k:8 * k + 8]
    u = s + pltpu.roll(s, 4, 0)
    w = u + pltpu.roll(u, 6, 0)
    r = w + pltpu.roll(w, 7, 0)
    sc = jnp.sqrt(r[0:1, :])
    key = jnp.where(m_ref[0, 0].reshape(1, tb) != 0,
                    lax.bitcast_convert_type(sc, jnp.int32),
                    jnp.int32(-1))
    keys_s[pl.ds(b, 1), pl.ds(ti * tb, tb)] = key

    @pl.when((b == nb - 1) & (ti == nt - 1))
    def _finalize():
        keys = keys_s[...]
        validf = jnp.sum((keys >= 0).astype(jnp.float32), axis=1, keepdims=True)
        kf = jnp.max(jnp.maximum(jnp.ceil(jnp.float32(_Q) * validf), 1.0))
        ki = kf.astype(jnp.int32)

        def bs_key(_, carry):
            lo, hi = carry
            mid = lo + (hi - lo) // 2
            cnt = jnp.sum((keys >= mid).astype(jnp.int32), axis=1, keepdims=True)
            ge = cnt >= ki
            return jnp.where(ge, mid, lo), jnp.where(ge, hi, mid)

        lo0 = jnp.full((nb, 1), -1, jnp.int32)
        hi0 = jnp.full((nb, 1), _INF_KEY, jnp.int32)
        thr, _ = lax.fori_loop(0, 31, bs_key, (lo0, hi0))

        c1 = jnp.sum((keys > thr).astype(jnp.int32), axis=1, keepdims=True)
        r = ki - c1
        tie = keys == thr
        iot = lax.broadcasted_iota(jnp.int32, (1, t), 1)

        def bs_idx(_, carry):
            lo, hi = carry
            mid = (lo + hi) // 2
            c = jnp.sum((tie & (iot < mid)).astype(jnp.int32), axis=1,
                        keepdims=True)
            ge = c >= r
            return jnp.where(ge, lo, mid + 1), jnp.where(ge, mid, hi)

        lo1 = jnp.zeros((nb, 1), jnp.int32)
        hi1 = jnp.full((nb, 1), t, jnp.int32)
        _, ibound = lax.fori_loop(0, 13, bs_idx, (lo1, hi1))

        sel = (keys > thr) | (tie & (iot < ibound))
        sel_ref[...] = sel.astype(jnp.int32).reshape(nb, 1, t)
        kf_ref[...] = jnp.full((1, 128), kf, jnp.float32)


def _sc_pool_kernel(h_ref, sel_ref, kf_ref, out_ref,
                    selv, idxv, rows, acc, parts, kfv, shared, sem,
                    *, t, d, ch):
    cid = lax.axis_index("c")
    sid = lax.axis_index("s")
    wid = cid * _NS + sid
    batch = wid // _CPB
    chunk = sid % _CPB
    row0 = pl.multiple_of(batch * t + chunk * ch, ch)
    nv = d // 16

    pltpu.sync_copy(sel_ref.at[pl.ds(row0, ch)], selv)
    pltpu.sync_copy(kf_ref, kfv)

    zf = jnp.zeros((16,), jnp.float32)
    for dv in range(nv):
        acc[pl.ds(dv * 16, 16)] = zf

    def cbody(v, off):
        lanes = selv[pl.ds(v * 16, 16)]
        m = lanes != jnp.zeros((16,), jnp.int32)
        mi = m.astype(jnp.int32)
        pos = jnp.full((16,), off - 1, jnp.int32) + plsc.cumsum(mi)
        gi = lax.iota(jnp.int32, 16) + jnp.full((16,), row0 + v * 16, jnp.int32)
        plsc.store_scatter(idxv, [pos], gi, mask=m)
        return off + jnp.sum(mi)

    cnt = lax.fori_loop(0, ch // 16, cbody, jnp.int32(0))

    plsc.store_scatter(idxv,
                       [lax.iota(jnp.int32, 16) + jnp.full((16,), cnt, jnp.int32)],
                       jnp.zeros((16,), jnp.int32))

    nch = (cnt + _G - 1) // _G

    def gbody(g, carry):
        gof = pl.multiple_of(g * _G, _G)
        pltpu.async_copy(h_ref.at[idxv.at[pl.ds(gof, _G)]], rows, sem).wait()
        for dv in range(nv):
            s = rows[0, pl.ds(dv * 16, 16)]
            for rr in range(1, _G):
                s = s + rows[rr, pl.ds(dv * 16, 16)]
            plsc.addupdate(acc.at[pl.ds(dv * 16, 16)], s)
        return carry

    lax.fori_loop(0, nch, gbody, jnp.int32(0))

    npad = nch * _G - cnt

    @pl.when(npad > 0)
    def _correct():
        npf = jnp.full((16,), -npad.astype(jnp.float32), jnp.float32)
        for dv in range(nv):
            plsc.addupdate(acc.at[pl.ds(dv * 16, 16)],
                           npf * rows[_G - 1, pl.ds(dv * 16, 16)])

    pltpu.sync_copy(acc, shared.at[pl.ds(pl.multiple_of(sid * d, d), d)])
    plsc.subcore_barrier()

    @pl.when(chunk == 0)
    def _reduce():
        pltpu.sync_copy(
            shared.at[pl.ds(pl.multiple_of(sid * d, d), _CPB * d)], parts)
        kvec = kfv[pl.ds(0, 16)]
        for dv in range(nv):
            s = parts[pl.ds(dv * 16, 16)]
            for rr in range(1, _CPB):
                s = s + parts[pl.ds(rr * d + dv * 16, 16)]
            acc[pl.ds(dv * 16, 16)] = s / kvec
        pltpu.sync_copy(
            acc, out_ref.at[pl.ds(pl.multiple_of(batch * d, d), d)])


def kernel(H, mask):
    B, T, D = H.shape
    tb = 512
    nt = T // tb
    ch = T // _CPB
    m3 = mask.astype(jnp.int32).reshape(B, 1, T)

    sel3, kfarr = pl.pallas_call(
        functools.partial(_score_select_kernel, nb=B, nt=nt, tb=tb, t=T),
        grid=(B, nt),
        in_specs=[
            pl.BlockSpec((1, tb, D), lambda b, ti: (b, ti, 0)),
            pl.BlockSpec((1, 1, tb), lambda b, ti: (b, 0, ti)),
        ],
        out_specs=[
            pl.BlockSpec((B, 1, T), lambda b, ti: (0, 0, 0)),
            pl.BlockSpec((1, 128), lambda b, ti: (0, 0)),
        ],
        out_shape=[
            jax.ShapeDtypeStruct((B, 1, T), jnp.int32),
            jax.ShapeDtypeStruct((1, 128), jnp.float32),
        ],
        scratch_shapes=[pltpu.VMEM((B, T), jnp.int32)],
    )(H, m3)

    mesh = plsc.VectorSubcoreMesh(core_axis_name="c", subcore_axis_name="s",
                                  num_cores=_NC, num_subcores=_NS)
    sc_pool = functools.partial(
        pl.kernel,
        mesh=mesh,
        compiler_params=pltpu.CompilerParams(needs_layout_passes=False),
        out_type=jax.ShapeDtypeStruct((B * D,), jnp.float32),
        scratch_types=[
            pltpu.VMEM((ch,), jnp.int32),
            pltpu.VMEM((ch + 16,), jnp.int32),
            pltpu.VMEM((_G, D), jnp.float32),
            pltpu.VMEM((D,), jnp.float32),
            pltpu.VMEM((_CPB * D,), jnp.float32),
            pltpu.VMEM((128,), jnp.float32),
            pltpu.VMEM_SHARED((_NS * D,), jnp.float32),
            pltpu.SemaphoreType.DMA,
        ],
    )(functools.partial(_sc_pool_kernel, t=T, d=D, ch=ch))

    out = sc_pool(H.reshape(B * T, D), sel3.reshape(B * T), kfarr.reshape(128))
    return out.reshape(B, D)


# speedup vs baseline: 1.2738x; 1.2738x over previous
import functools

import jax
import jax.numpy as jnp
from jax import lax
from jax.experimental import pallas as pl
from jax.experimental.pallas import tpu as pltpu
from jax.experimental.pallas import tpu_sc as plsc

_Q = 0.15
_INF_KEY = 2139095041
_NC = 2
_NS = 16
_CPB = 8
_G = 8


def _score_select_kernel(h_ref, m_ref, sel_ref, kf_ref, keys_s, *, nb, nt, tb, t):
    b = pl.program_id(0)
    ti = pl.program_id(1)
    x = h_ref[0]
    d = x.shape[1]
    y = x * x
    acc = y[:, 0:128]
    for i in range(1, d // 128):
        acc = acc + y[:, i * 128:(i + 1) * 128]
    v = acc.T
    s = v[0:8]
    for k in range(1, 16):
        s = s + v[8 * k:8 * k + 8]
    u = s + pltpu.roll(s, 4, 0)
    w = u + pltpu.roll(u, 6, 0)
    r = w + pltpu.roll(w, 7, 0)
    sc = jnp.sqrt(r[0:1, :])
    key = jnp.where(m_ref[0, 0].reshape(1, tb) != 0,
                    lax.bitcast_convert_type(sc, jnp.int32),
                    jnp.int32(-1))
    keys_s[pl.ds(b, 1), pl.ds(ti * tb, tb)] = key

    @pl.when((b == nb - 1) & (ti == nt - 1))
    def _finalize():
        keys = keys_s[...]
        validf = jnp.sum((keys >= 0).astype(jnp.float32), axis=1, keepdims=True)
        kf = jnp.max(jnp.maximum(jnp.ceil(jnp.float32(_Q) * validf), 1.0))
        ki = kf.astype(jnp.int32)

        def bs_key(_, carry):
            lo, hi = carry
            mid = lo + (hi - lo) // 2
            cnt = jnp.sum((keys >= mid).astype(jnp.int32), axis=1, keepdims=True)
            ge = cnt >= ki
            return jnp.where(ge, mid, lo), jnp.where(ge, hi, mid)

        lo0 = jnp.full((nb, 1), -1, jnp.int32)
        hi0 = jnp.full((nb, 1), _INF_KEY, jnp.int32)
        thr, _ = lax.fori_loop(0, 31, bs_key, (lo0, hi0))

        c1 = jnp.sum((keys > thr).astype(jnp.int32), axis=1, keepdims=True)
        r = ki - c1
        tie = keys == thr
        iot = lax.broadcasted_iota(jnp.int32, (1, t), 1)

        def bs_idx(_, carry):
            lo, hi = carry
            mid = (lo + hi) // 2
            c = jnp.sum((tie & (iot < mid)).astype(jnp.int32), axis=1,
                        keepdims=True)
            ge = c >= r
            return jnp.where(ge, lo, mid + 1), jnp.where(ge, mid, hi)

        lo1 = jnp.zeros((nb, 1), jnp.int32)
        hi1 = jnp.full((nb, 1), t, jnp.int32)
        _, ibound = lax.fori_loop(0, 13, bs_idx, (lo1, hi1))

        sel = (keys > thr) | (tie & (iot < ibound))
        sel_ref[...] = sel.astype(jnp.int32).reshape(nb, 1, t)
        kf_ref[...] = jnp.full((1, 128), kf, jnp.float32)


def _sc_pool_kernel(h_ref, sel_ref, kf_ref, out_ref,
                    selv, idxv, rows, rows2, acc, parts, kfv, shared,
                    sem, sem2, *, t, d, ch):
    cid = lax.axis_index("c")
    sid = lax.axis_index("s")
    wid = cid * _NS + sid
    batch = wid // _CPB
    chunk = sid % _CPB
    row0 = pl.multiple_of(batch * t + chunk * ch, ch)
    nv = d // 16

    pltpu.sync_copy(sel_ref.at[pl.ds(row0, ch)], selv)
    pltpu.sync_copy(kf_ref, kfv)

    zf = jnp.zeros((16,), jnp.float32)
    for dv in range(nv):
        acc[pl.ds(dv * 16, 16)] = zf

    def cbody(v, off):
        lanes = selv[pl.ds(v * 16, 16)]
        m = lanes != jnp.zeros((16,), jnp.int32)
        mi = m.astype(jnp.int32)
        pos = jnp.full((16,), off - 1, jnp.int32) + plsc.cumsum(mi)
        gi = lax.iota(jnp.int32, 16) + jnp.full((16,), row0 + v * 16, jnp.int32)
        plsc.store_scatter(idxv, [pos], gi, mask=m)
        return off + jnp.sum(mi)

    cnt = lax.fori_loop(0, ch // 16, cbody, jnp.int32(0))

    plsc.store_scatter(idxv,
                       [lax.iota(jnp.int32, 16) + jnp.full((16,), cnt, jnp.int32)],
                       jnp.zeros((16,), jnp.int32))

    nch = (cnt + _G - 1) // _G

    def _start(buf, sm, g):
        gof = pl.multiple_of(g * _G, _G)
        pltpu.async_copy(h_ref.at[idxv.at[pl.ds(gof, _G)]], buf, sm)

    def _wait(buf, sm):
        pltpu.make_async_copy(h_ref.at[idxv.at[pl.ds(0, _G)]], buf, sm).wait()

    def _accum(buf):
        def dvbody(i, carry):
            o = pl.multiple_of(i * 128, 128)
            for k in range(8):
                dvo = o + k * 16
                s = buf[0, pl.ds(dvo, 16)]
                for rr in range(1, _G):
                    s = s + buf[rr, pl.ds(dvo, 16)]
                plsc.addupdate(acc.at[pl.ds(dvo, 16)], s)
            return carry
        lax.fori_loop(0, nv // 8, dvbody, jnp.int32(0))

    @pl.when(nch > 0)
    def _prime():
        _start(rows, sem, jnp.int32(0))

    def gbody(g, carry):
        even = lax.rem(g, 2) == 0

        @pl.when(g + 1 < nch)
        def _next():
            @pl.when(even)
            def _():
                _start(rows2, sem2, g + 1)

            @pl.when(jnp.logical_not(even))
            def _():
                _start(rows, sem, g + 1)

        @pl.when(even)
        def _even():
            _wait(rows, sem)
            _accum(rows)

        @pl.when(jnp.logical_not(even))
        def _odd():
            _wait(rows2, sem2)
            _accum(rows2)

        return carry

    lax.fori_loop(0, nch, gbody, jnp.int32(0))

    npad = nch * _G - cnt

    @pl.when(npad > 0)
    def _correct():
        npf = jnp.full((16,), -npad.astype(jnp.float32), jnp.float32)
        lastbuf_even = lax.rem(nch - 1, 2) == 0

        def corr(buf):
            def dvbody(i, carry):
                o = pl.multiple_of(i * 128, 128)
                for k in range(8):
                    dvo = o + k * 16
                    plsc.addupdate(acc.at[pl.ds(dvo, 16)],
                                   npf * buf[_G - 1, pl.ds(dvo, 16)])
                return carry
            lax.fori_loop(0, nv // 8, dvbody, jnp.int32(0))

        @pl.when(lastbuf_even)
        def _():
            corr(rows)

        @pl.when(jnp.logical_not(lastbuf_even))
        def _():
            corr(rows2)

    pltpu.sync_copy(acc, shared.at[pl.ds(pl.multiple_of(sid * d, d), d)])
    plsc.subcore_barrier()

    @pl.when(chunk == 0)
    def _reduce():
        pltpu.sync_copy(
            shared.at[pl.ds(pl.multiple_of(sid * d, d), _CPB * d)], parts)
        kvec = kfv[pl.ds(0, 16)]
        for dv in range(nv):
            s = parts[pl.ds(dv * 16, 16)]
            for rr in range(1, _CPB):
                s = s + parts[pl.ds(rr * d + dv * 16, 16)]
            acc[pl.ds(dv * 16, 16)] = s / kvec
        pltpu.sync_copy(
            acc, out_ref.at[pl.ds(pl.multiple_of(batch * d, d), d)])


def kernel(H, mask):
    B, T, D = H.shape
    tb = 512
    nt = T // tb
    ch = T // _CPB
    m3 = mask.astype(jnp.int32).reshape(B, 1, T)

    sel3, kfarr = pl.pallas_call(
        functools.partial(_score_select_kernel, nb=B, nt=nt, tb=tb, t=T),
        grid=(B, nt),
        in_specs=[
            pl.BlockSpec((1, tb, D), lambda b, ti: (b, ti, 0)),
            pl.BlockSpec((1, 1, tb), lambda b, ti: (b, 0, ti)),
        ],
        out_specs=[
            pl.BlockSpec((B, 1, T), lambda b, ti: (0, 0, 0)),
            pl.BlockSpec((1, 128), lambda b, ti: (0, 0)),
        ],
        out_shape=[
            jax.ShapeDtypeStruct((B, 1, T), jnp.int32),
            jax.ShapeDtypeStruct((1, 128), jnp.float32),
        ],
        scratch_shapes=[pltpu.VMEM((B, T), jnp.int32)],
    )(H, m3)

    mesh = plsc.VectorSubcoreMesh(core_axis_name="c", subcore_axis_name="s",
                                  num_cores=_NC, num_subcores=_NS)
    sc_pool = functools.partial(
        pl.kernel,
        mesh=mesh,
        compiler_params=pltpu.CompilerParams(needs_layout_passes=False),
        out_type=jax.ShapeDtypeStruct((B * D,), jnp.float32),
        scratch_types=[
            pltpu.VMEM((ch,), jnp.int32),
            pltpu.VMEM((ch + 16,), jnp.int32),
            pltpu.VMEM((_G, D), jnp.float32),
            pltpu.VMEM((_G, D), jnp.float32),
            pltpu.VMEM((D,), jnp.float32),
            pltpu.VMEM((_CPB * D,), jnp.float32),
            pltpu.VMEM((128,), jnp.float32),
            pltpu.VMEM_SHARED((_NS * D,), jnp.float32),
            pltpu.SemaphoreType.DMA,
            pltpu.SemaphoreType.DMA,
        ],
    )(functools.partial(_sc_pool_kernel, t=T, d=D, ch=ch))

    out = sc_pool(H.reshape(B * T, D), sel3.reshape(B * T), kfarr.reshape(128))
    return out.reshape(B, D)


# speedup vs baseline: 1.4005x; 1.0995x over previous
import functools

import jax
import jax.numpy as jnp
from jax import lax
from jax.experimental import pallas as pl
from jax.experimental.pallas import tpu as pltpu
from jax.experimental.pallas import tpu_sc as plsc

_Q = 0.15
_INF_KEY = 2139095041
_NC = 2
_NS = 16
_CPB = 8
_G = 8


def _score_select_kernel(h_ref, m_ref, sel_ref, kf_ref, keys_s, *, nb, nt, tb, t):
    b = pl.program_id(0)
    ti = pl.program_id(1)
    x = h_ref[0]
    d = x.shape[1]
    y = x * x
    acc = y[:, 0:128]
    for i in range(1, d // 128):
        acc = acc + y[:, i * 128:(i + 1) * 128]
    v = acc.T
    s = v[0:8]
    for k in range(1, 16):
        s = s + v[8 * k:8 * k + 8]
    u = s + pltpu.roll(s, 4, 0)
    w = u + pltpu.roll(u, 6, 0)
    r = w + pltpu.roll(w, 7, 0)
    sc = jnp.sqrt(r[0:1, :])
    key = jnp.where(m_ref[0, 0].reshape(1, tb) != 0,
                    lax.bitcast_convert_type(sc, jnp.int32),
                    jnp.int32(-1))
    keys_s[pl.ds(b, 1), pl.ds(ti * tb, tb)] = key

    @pl.when((b == nb - 1) & (ti == nt - 1))
    def _finalize():
        keys = keys_s[...]
        validf = jnp.sum((keys >= 0).astype(jnp.float32), axis=1, keepdims=True)
        kf = jnp.max(jnp.maximum(jnp.ceil(jnp.float32(_Q) * validf), 1.0))
        ki = kf.astype(jnp.int32)

        def bs_key(_, carry):
            lo, hi = carry
            mid = lo + (hi - lo) // 2
            cnt = jnp.sum((keys >= mid).astype(jnp.int32), axis=1, keepdims=True)
            ge = cnt >= ki
            return jnp.where(ge, mid, lo), jnp.where(ge, hi, mid)

        lo0 = jnp.full((nb, 1), -1, jnp.int32)
        hi0 = jnp.full((nb, 1), _INF_KEY, jnp.int32)
        thr, _ = lax.fori_loop(0, 31, bs_key, (lo0, hi0))

        c1 = jnp.sum((keys > thr).astype(jnp.int32), axis=1, keepdims=True)
        r = ki - c1
        tie = keys == thr
        iot = lax.broadcasted_iota(jnp.int32, (1, t), 1)

        def bs_idx(_, carry):
            lo, hi = carry
            mid = (lo + hi) // 2
            c = jnp.sum((tie & (iot < mid)).astype(jnp.int32), axis=1,
                        keepdims=True)
            ge = c >= r
            return jnp.where(ge, lo, mid + 1), jnp.where(ge, mid, hi)

        lo1 = jnp.zeros((nb, 1), jnp.int32)
        hi1 = jnp.full((nb, 1), t, jnp.int32)
        _, ibound = lax.fori_loop(0, 13, bs_idx, (lo1, hi1))

        sel = (keys > thr) | (tie & (iot < ibound))
        sel_ref[...] = sel.astype(jnp.int32).reshape(nb, 1, t)
        kf_ref[...] = jnp.full((1, 128), kf, jnp.float32)


def _sc_pool_kernel(h_ref, sel_ref, kf_ref, out_ref,
                    selv, idxv, rows, rows2, acc, parts, kfv, shared,
                    sem, sem2, *, t, d, ch):
    cid = lax.axis_index("c")
    sid = lax.axis_index("s")
    wid = cid * _NS + sid
    batch = wid // _CPB
    chunk = sid % _CPB
    row0 = pl.multiple_of(batch * t + chunk * ch, ch)
    nv = d // 16

    pltpu.sync_copy(sel_ref.at[pl.ds(row0, ch)], selv)
    pltpu.sync_copy(kf_ref, kfv)

    zf = jnp.zeros((16,), jnp.float32)
    for dv in range(nv):
        acc[pl.ds(dv * 16, 16)] = zf

    def cbody(v, off):
        lanes = selv[pl.ds(v * 16, 16)]
        m = lanes != jnp.zeros((16,), jnp.int32)
        mi = m.astype(jnp.int32)
        pos = jnp.full((16,), off - 1, jnp.int32) + plsc.cumsum(mi)
        gi = lax.iota(jnp.int32, 16) + jnp.full((16,), row0 + v * 16, jnp.int32)
        plsc.store_scatter(idxv, [pos], gi, mask=m)
        return off + jnp.sum(mi)

    cnt = lax.fori_loop(0, ch // 16, cbody, jnp.int32(0))

    plsc.store_scatter(idxv,
                       [lax.iota(jnp.int32, 16) + jnp.full((16,), cnt, jnp.int32)],
                       jnp.zeros((16,), jnp.int32))

    nch = (cnt + _G - 1) // _G

    def _start(buf, sm, g):
        gof = pl.multiple_of(g * _G, _G)
        pltpu.async_copy(h_ref.at[idxv.at[pl.ds(gof, _G)]], buf, sm)

    def _wait(buf, sm):
        pltpu.make_async_copy(h_ref.at[idxv.at[pl.ds(0, _G)]], buf, sm).wait()

    def _accum(buf):
        def dvbody(i, carry):
            o = pl.multiple_of(i * 128, 128)
            for k in range(8):
                dvo = o + k * 16
                s = buf[0, pl.ds(dvo, 16)]
                for rr in range(1, _G):
                    s = s + buf[rr, pl.ds(dvo, 16)]
                plsc.addupdate(acc.at[pl.ds(dvo, 16)], s)
            return carry
        lax.fori_loop(0, nv // 8, dvbody, jnp.int32(0))

    @pl.when(nch > 0)
    def _prime():
        _start(rows, sem, jnp.int32(0))

    def gbody(g, carry):
        even = lax.rem(g, 2) == 0

        @pl.when(g + 1 < nch)
        def _next():
            @pl.when(even)
            def _():
                _start(rows2, sem2, g + 1)

            @pl.when(jnp.logical_not(even))
            def _():
                _start(rows, sem, g + 1)

        @pl.when(even)
        def _even():
            _wait(rows, sem)
            _accum(rows)

        @pl.when(jnp.logical_not(even))
        def _odd():
            _wait(rows2, sem2)
            _accum(rows2)

        return carry

    lax.fori_loop(0, nch, gbody, jnp.int32(0))

    npad = nch * _G - cnt

    @pl.when(npad > 0)
    def _correct():
        npf = jnp.full((16,), -npad.astype(jnp.float32), jnp.float32)
        lastbuf_even = lax.rem(nch - 1, 2) == 0

        def corr(buf):
            def dvbody(i, carry):
                o = pl.multiple_of(i * 128, 128)
                for k in range(8):
                    dvo = o + k * 16
                    plsc.addupdate(acc.at[pl.ds(dvo, 16)],
                                   npf * buf[_G - 1, pl.ds(dvo, 16)])
                return carry
            lax.fori_loop(0, nv // 8, dvbody, jnp.int32(0))

        @pl.when(lastbuf_even)
        def _():
            corr(rows)

        @pl.when(jnp.logical_not(lastbuf_even))
        def _():
            corr(rows2)

    pltpu.sync_copy(acc, shared.at[pl.ds(pl.multiple_of(sid * d, d), d)])
    plsc.subcore_barrier()

    ds8 = d // _CPB
    dsl = chunk * ds8
    gb = (sid // _CPB) * _CPB
    for rr in range(_CPB):
        pltpu.async_copy(
            shared.at[pl.ds(pl.multiple_of((gb + rr) * d + dsl, ds8), ds8)],
            parts.at[pl.ds(rr * ds8, ds8)], sem)
    for rr in range(_CPB):
        pltpu.make_async_copy(shared.at[pl.ds(0, ds8)],
                              parts.at[pl.ds(rr * ds8, ds8)], sem).wait()
    kvec = kfv[pl.ds(0, 16)]
    for dv in range(ds8 // 16):
        s = parts[pl.ds(dv * 16, 16)]
        for rr in range(1, _CPB):
            s = s + parts[pl.ds(rr * ds8 + dv * 16, 16)]
        acc[pl.ds(dv * 16, 16)] = s / kvec
    pltpu.sync_copy(acc.at[pl.ds(0, ds8)],
                    out_ref.at[pl.ds(pl.multiple_of(batch * d + dsl, ds8),
                                     ds8)])


def kernel(H, mask):
    B, T, D = H.shape
    tb = 1024
    nt = T // tb
    ch = T // _CPB
    m3 = mask.astype(jnp.int32).reshape(B, 1, T)

    sel3, kfarr = pl.pallas_call(
        functools.partial(_score_select_kernel, nb=B, nt=nt, tb=tb, t=T),
        grid=(B, nt),
        in_specs=[
            pl.BlockSpec((1, tb, D), lambda b, ti: (b, ti, 0)),
            pl.BlockSpec((1, 1, tb), lambda b, ti: (b, 0, ti)),
        ],
        out_specs=[
            pl.BlockSpec((B, 1, T), lambda b, ti: (0, 0, 0)),
            pl.BlockSpec((1, 128), lambda b, ti: (0, 0)),
        ],
        out_shape=[
            jax.ShapeDtypeStruct((B, 1, T), jnp.int32),
            jax.ShapeDtypeStruct((1, 128), jnp.float32),
        ],
        scratch_shapes=[pltpu.VMEM((B, T), jnp.int32)],
    )(H, m3)

    mesh = plsc.VectorSubcoreMesh(core_axis_name="c", subcore_axis_name="s",
                                  num_cores=_NC, num_subcores=_NS)
    sc_pool = functools.partial(
        pl.kernel,
        mesh=mesh,
        compiler_params=pltpu.CompilerParams(needs_layout_passes=False),
        out_type=jax.ShapeDtypeStruct((B * D,), jnp.float32),
        scratch_types=[
            pltpu.VMEM((ch,), jnp.int32),
            pltpu.VMEM((ch + 16,), jnp.int32),
            pltpu.VMEM((_G, D), jnp.float32),
            pltpu.VMEM((_G, D), jnp.float32),
            pltpu.VMEM((D,), jnp.float32),
            pltpu.VMEM((D,), jnp.float32),
            pltpu.VMEM((128,), jnp.float32),
            pltpu.VMEM_SHARED((_NS * D,), jnp.float32),
            pltpu.SemaphoreType.DMA,
            pltpu.SemaphoreType.DMA,
        ],
    )(functools.partial(_sc_pool_kernel, t=T, d=D, ch=ch))

    out = sc_pool(H.reshape(B * T, D), sel3.reshape(B * T), kfarr.reshape(128))
    return out.reshape(B, D)
